# Initial kernel scaffold; baseline (speedup 1.0000x reference)
import jax
import jax.numpy as jnp
from jax.experimental import pallas as pl


def kernel(logits):
    raise NotImplementedError("write your pallas kernel here")


# speedup vs baseline: 19.1045x; 19.1045x over previous
import functools

import jax
import jax.numpy as jnp
from jax import lax
from jax.experimental import pallas as pl
from jax.experimental.pallas import tpu as pltpu
from jax.experimental.pallas import tpu_sc as plsc

ROWS = 128
COLS = 32768
L = 16
SLICES = COLS // L
UNROLL = 8
NC = 2
NS = 16
NW = NC * NS
ROWS_PER = ROWS // NW


def _sparsemax_body(logits_hbm, out_hbm, buf):
    wid = lax.axis_index("s") * NC + lax.axis_index("c")

    def do_row(r, carry):
        row = wid * ROWS_PER + r
        pltpu.sync_copy(logits_hbm.at[row], buf)

        def max_body(i, acc):
            for j in range(UNROLL):
                acc = jnp.maximum(acc, buf[pl.ds((i * UNROLL + j) * L, L)])
            return acc
        acc0 = jnp.full((L,), -jnp.inf, dtype=jnp.float32)
        colmax = lax.fori_loop(0, SLICES // UNROLL, max_body, acc0)
        m = jnp.max(colmax)
        t0 = jnp.broadcast_to(m, (L,)) - 1.0

        def sum_count(tvec):
            def body(i, carry):
                s, k = carry
                for j in range(UNROLL):
                    v = buf[pl.ds((i * UNROLL + j) * L, L)]
                    mask = v > tvec
                    s = s + jnp.where(mask, v, 0.0)
                    k = k + jnp.where(mask, 1.0, 0.0)
                return s, k
            z16 = jnp.zeros((L,), dtype=jnp.float32)
            s, k = lax.fori_loop(0, SLICES // UNROLL, body, (z16, z16))
            return jnp.sum(s), jnp.sum(k)

        def newton_cond(c):
            _, done = c
            return jnp.logical_not(done)

        def newton_step(c):
            t, _ = c
            s, k = sum_count(t)
            t_new = (jnp.broadcast_to(s, (L,)) - 1.0) / jnp.broadcast_to(k, (L,))
            t_up = jnp.maximum(t, t_new)
            done = jnp.all(t_up == t)
            return t_up, done

        tau, _ = lax.while_loop(newton_cond, newton_step, (t0, False))

        def relu_body(i, carry):
            for j in range(UNROLL):
                idx = pl.ds((i * UNROLL + j) * L, L)
                buf[idx] = jnp.maximum(buf[idx] - tau, 0.0)
            return carry
        lax.fori_loop(0, SLICES // UNROLL, relu_body, 0)

        pltpu.sync_copy(buf, out_hbm.at[row])
        return carry

    lax.fori_loop(0, ROWS_PER, do_row, 0)


@jax.jit
def _sparsemax_sc(logits):
    mesh = plsc.VectorSubcoreMesh(core_axis_name="c", subcore_axis_name="s")
    kfn = functools.partial(
        pl.kernel,
        mesh=mesh,
        out_type=jax.ShapeDtypeStruct((ROWS, COLS), jnp.float32),
        scratch_types=[pltpu.VMEM((COLS,), jnp.float32)],
        compiler_params=pltpu.CompilerParams(needs_layout_passes=False),
    )(_sparsemax_body)
    return kfn(logits)


def kernel(logits):
    return _sparsemax_sc(logits.astype(jnp.float32))
